# baseline (speedup 1.0000x reference)
import functools
import math

import jax
import jax.numpy as jnp
from jax import lax
from jax.experimental import pallas as pl
from jax.experimental.pallas import tpu as pltpu
from jax.experimental.pallas import tpu_sc as plsc

D_MODEL = 64
SCALE = math.sqrt(D_MODEL)
NUM_CORES = 2
NUM_SUBCORES = 16
NUM_WORKERS = NUM_CORES * NUM_SUBCORES
LANES = 16
SEQ = 200
BTOK = 128
DG = D_MODEL // 8
NBUF = 2


def _transpose_tc(x):
    n_rows, seq = x.shape
    blk = 256

    def body(x_ref, o_ref):
        o_ref[...] = x_ref[...].T

    return pl.pallas_call(
        body,
        out_shape=jax.ShapeDtypeStruct((seq, n_rows), jnp.int32),
        grid=(n_rows // blk,),
        in_specs=[pl.BlockSpec((blk, seq), lambda i: (i, 0))],
        out_specs=pl.BlockSpec((seq, blk), lambda i: (0, i)),
    )(x)


def _emb_body(xt_hbm, table_hbm, out_hbm, *scratch):
    idx_v = scratch[:NBUF]
    rows_v = scratch[NBUF:2 * NBUF]
    trans_v = scratch[2 * NBUF:3 * NBUF]
    gsem = scratch[3 * NBUF:4 * NBUF]
    ssem = scratch[4 * NBUF:5 * NBUF]

    w = lax.axis_index("s") * NUM_CORES + lax.axis_index("c")
    b0 = w * BTOK
    iota = lax.iota(jnp.int32, LANES)

    for b in range(NBUF):
        pltpu.sync_copy(xt_hbm.at[b, pl.ds(b0, BTOK)], idx_v[b])
        pltpu.async_copy(table_hbm.at[idx_v[b]], rows_v[b], gsem[b])

    def super_body(k, carry):
        for b in range(NBUF):
            t = k * NBUF + b
            pltpu.make_async_copy(table_hbm.at[idx_v[b]], rows_v[b],
                                  gsem[b]).wait()

            @pl.when(t >= NBUF)
            def _():
                for dg in range(DG):
                    pltpu.make_async_copy(
                        trans_v[b].at[pl.ds(8 * dg, 8)],
                        out_hbm.at[t - NBUF, dg, w], ssem[b]).wait()

            @plsc.parallel_loop(0, D_MODEL, step=1, unroll=4)
            def _tr(d):
                dvec = jnp.full((LANES,), 0, jnp.int32) + d
                for g in range(BTOK // LANES):
                    bvec = iota + (g * LANES)
                    v = plsc.load_gather(rows_v[b], [bvec, dvec])
                    trans_v[b][d, pl.ds(g * LANES, LANES)] = v * SCALE

            for dg in range(DG):
                pltpu.async_copy(trans_v[b].at[pl.ds(8 * dg, 8)],
                                 out_hbm.at[t, dg, w], ssem[b])

            nxt = t + NBUF

            @pl.when(nxt < SEQ)
            def _():
                pltpu.sync_copy(xt_hbm.at[nxt, pl.ds(b0, BTOK)], idx_v[b])
                pltpu.async_copy(table_hbm.at[idx_v[b]], rows_v[b], gsem[b])

        return carry

    lax.fori_loop(0, SEQ // NBUF, super_body, 0)

    for b in range(NBUF):
        t = SEQ - NBUF + b
        for dg in range(DG):
            pltpu.make_async_copy(trans_v[b].at[pl.ds(8 * dg, 8)],
                                  out_hbm.at[t, dg, w], ssem[b]).wait()


def kernel(x, table):
    n_rows, seq = x.shape
    assert seq == SEQ and n_rows == NUM_WORKERS * BTOK

    xt = _transpose_tc(x)

    mesh = plsc.VectorSubcoreMesh(
        core_axis_name="c", subcore_axis_name="s",
        num_cores=NUM_CORES, num_subcores=NUM_SUBCORES,
    )
    scratch = (
        [pltpu.VMEM((BTOK,), jnp.int32) for _ in range(NBUF)]
        + [pltpu.VMEM((BTOK, D_MODEL), jnp.float32) for _ in range(NBUF)]
        + [pltpu.VMEM((D_MODEL, BTOK), jnp.float32) for _ in range(NBUF)]
        + [pltpu.SemaphoreType.DMA for _ in range(2 * NBUF)]
    )
    f = functools.partial(
        pl.kernel,
        out_type=jax.ShapeDtypeStruct(
            (SEQ, DG, NUM_WORKERS, 8, BTOK), jnp.float32),
        mesh=mesh,
        scratch_types=scratch,
        compiler_params=pltpu.CompilerParams(use_tc_tiling_on_sc=False,
                                             needs_layout_passes=False),
    )(_emb_body)
    out5 = f(xt, table)
    return out5.transpose(2, 4, 0, 1, 3).reshape(n_rows, SEQ, D_MODEL)


# speedup vs baseline: 1.0365x; 1.0365x over previous
import functools
import math

import jax
import jax.numpy as jnp
from jax import lax
from jax.experimental import pallas as pl
from jax.experimental.pallas import tpu as pltpu
from jax.experimental.pallas import tpu_sc as plsc

D_MODEL = 64
SCALE = math.sqrt(D_MODEL)
NUM_CORES = 2
NUM_SUBCORES = 16
NUM_WORKERS = NUM_CORES * NUM_SUBCORES
LANES = 16
SEQ = 200
BTOK = 128
DG = D_MODEL // 8
NBUF = 2


def _emb_body(xt_hbm, table_hbm, out_hbm, idxblk_v, *scratch):
    rows_v = scratch[:NBUF]
    trans_v = scratch[NBUF:2 * NBUF]
    isem = scratch[2 * NBUF]
    gsem = scratch[2 * NBUF + 1:2 * NBUF + 1 + NBUF]
    ssem = scratch[2 * NBUF + 1 + NBUF:2 * NBUF + 1 + 2 * NBUF]

    w = lax.axis_index("s") * NUM_CORES + lax.axis_index("c")
    b0 = w * BTOK
    iota = lax.iota(jnp.int32, LANES)

    pltpu.async_copy(xt_hbm.at[:, pl.ds(b0, BTOK)], idxblk_v, isem).wait()

    for b in range(NBUF):
        pltpu.async_copy(table_hbm.at[idxblk_v.at[b]], rows_v[b], gsem[b])

    def super_body(k, carry):
        for b in range(NBUF):
            t = k * NBUF + b
            pltpu.make_async_copy(table_hbm.at[idxblk_v.at[t]], rows_v[b],
                                  gsem[b]).wait()

            @pl.when(t >= NBUF)
            def _():
                for dg in range(DG):
                    pltpu.make_async_copy(
                        trans_v[b].at[pl.ds(8 * dg, 8)],
                        out_hbm.at[t - NBUF, dg, w], ssem[b]).wait()

            @plsc.parallel_loop(0, D_MODEL, step=1, unroll=8)
            def _tr(d):
                dvec = iota * 0 + d
                for g in range(BTOK // LANES):
                    bvec = iota + (g * LANES)
                    v = plsc.load_gather(rows_v[b], [bvec, dvec])
                    trans_v[b][d, pl.ds(g * LANES, LANES)] = v * SCALE

            for dg in range(DG):
                pltpu.async_copy(trans_v[b].at[pl.ds(8 * dg, 8)],
                                 out_hbm.at[t, dg, w], ssem[b])

            nxt = t + NBUF

            @pl.when(nxt < SEQ)
            def _():
                pltpu.async_copy(table_hbm.at[idxblk_v.at[nxt]], rows_v[b],
                                 gsem[b])

        return carry

    lax.fori_loop(0, SEQ // NBUF, super_body, 0)

    for b in range(NBUF):
        t = SEQ - NBUF + b
        for dg in range(DG):
            pltpu.make_async_copy(trans_v[b].at[pl.ds(8 * dg, 8)],
                                  out_hbm.at[t, dg, w], ssem[b]).wait()


def kernel(x, table):
    n_rows, seq = x.shape
    assert seq == SEQ and n_rows == NUM_WORKERS * BTOK

    xt = x.T

    mesh = plsc.VectorSubcoreMesh(
        core_axis_name="c", subcore_axis_name="s",
        num_cores=NUM_CORES, num_subcores=NUM_SUBCORES,
    )
    scratch = (
        [pltpu.VMEM((SEQ, BTOK), jnp.int32)]
        + [pltpu.VMEM((BTOK, D_MODEL), jnp.float32) for _ in range(NBUF)]
        + [pltpu.VMEM((D_MODEL, BTOK), jnp.float32) for _ in range(NBUF)]
        + [pltpu.SemaphoreType.DMA for _ in range(1 + 2 * NBUF)]
    )
    f = functools.partial(
        pl.kernel,
        out_type=jax.ShapeDtypeStruct(
            (SEQ, DG, NUM_WORKERS, 8, BTOK), jnp.float32),
        mesh=mesh,
        scratch_types=scratch,
        compiler_params=pltpu.CompilerParams(use_tc_tiling_on_sc=False,
                                             needs_layout_passes=False),
    )(_emb_body)
    out5 = f(xt, table)
    return out5.transpose(2, 4, 0, 1, 3).reshape(n_rows, SEQ, D_MODEL)
